# Initial kernel scaffold; baseline (speedup 1.0000x reference)
import jax
import jax.numpy as jnp
from jax.experimental import pallas as pl


def kernel(x, emb, W1, b1, W2, b2):
    raise NotImplementedError("write your pallas kernel here")


# speedup vs baseline: 1.0441x; 1.0441x over previous
import functools

import jax
import jax.numpy as jnp
from jax import lax
from jax.experimental import pallas as pl
from jax.experimental.pallas import tpu as pltpu
from jax.experimental.pallas import tpu_sc as plsc

VOCAB = 1000000
EMBED_DIM = 64
BATCH = 4096
SEQ_LEN = 200

NC = 2
NS = 16
NW = NC * NS
B_PER_W = BATCH // NW
HALF = SEQ_LEN // 2


def _sum_rows(rows_ref, acc):
    def body(t, acc):
        a0, a1, a2, a3 = acc
        a0 = a0 + rows_ref[t, pl.ds(0, 16)]
        a1 = a1 + rows_ref[t, pl.ds(16, 16)]
        a2 = a2 + rows_ref[t, pl.ds(32, 16)]
        a3 = a3 + rows_ref[t, pl.ds(48, 16)]
        return (a0, a1, a2, a3)
    return lax.fori_loop(0, HALF, body, acc, unroll=2)


def _sc_pool_sums(x3, emb):
    mesh = plsc.VectorSubcoreMesh(core_axis_name="c", subcore_axis_name="s")

    @functools.partial(
        pl.kernel,
        out_type=jax.ShapeDtypeStruct((BATCH, EMBED_DIM), jnp.float32),
        mesh=mesh,
        scratch_types=[
            pltpu.VMEM((B_PER_W, 2, HALF), jnp.int32),
            pltpu.VMEM((2, 2, HALF, EMBED_DIM), jnp.float32),
            pltpu.VMEM((B_PER_W, EMBED_DIM), jnp.float32),
            pltpu.SemaphoreType.DMA,
            pltpu.SemaphoreType.DMA,
        ],
        compiler_params=pltpu.CompilerParams(use_tc_tiling_on_sc=False),
    )
    def k(x_hbm, emb_hbm, out_hbm, idx_v, rows_v, acc_v, sem0, sem1):
        wid = lax.axis_index("s") * NC + lax.axis_index("c")
        base = wid * B_PER_W
        pltpu.sync_copy(x_hbm.at[pl.ds(base, B_PER_W)], idx_v)

        sems = (sem0, sem1)

        def fire(b, buf):
            pltpu.async_copy(emb_hbm.at[idx_v.at[b, 0]], rows_v.at[buf, 0], sems[buf])
            pltpu.async_copy(emb_hbm.at[idx_v.at[b, 1]], rows_v.at[buf, 1], sems[buf])

        def drain(b, buf):
            pltpu.make_async_copy(emb_hbm.at[idx_v.at[b, 0]], rows_v.at[buf, 0], sems[buf]).wait()
            pltpu.make_async_copy(emb_hbm.at[idx_v.at[b, 1]], rows_v.at[buf, 1], sems[buf]).wait()

        def consume(b, buf):
            drain(b, buf)
            zeros = jnp.zeros((16,), jnp.float32)
            acc = (zeros, zeros, zeros, zeros)
            acc = _sum_rows(rows_v.at[buf, 0], acc)
            acc = _sum_rows(rows_v.at[buf, 1], acc)
            a0, a1, a2, a3 = acc
            acc_v[b, pl.ds(0, 16)] = a0
            acc_v[b, pl.ds(16, 16)] = a1
            acc_v[b, pl.ds(32, 16)] = a2
            acc_v[b, pl.ds(48, 16)] = a3

        fire(0, 0)

        def body(g, _):
            b = g * 2
            fire(b + 1, 1)
            consume(b, 0)

            @pl.when(b + 2 < B_PER_W)
            def _():
                fire(b + 2, 0)

            consume(b + 1, 1)
            return 0

        lax.fori_loop(0, B_PER_W // 2, body, 0)
        pltpu.sync_copy(acc_v, out_hbm.at[pl.ds(base, B_PER_W)])

    return k(x3, emb)


def _tc_head(x, sums, W1, b1, W2, b2):
    def body(x_ref, sums_ref, W1_ref, b1_ref, W2_ref, b2_ref, out_ref):
        cnt = jnp.sum((x_ref[...] != 0).astype(jnp.float32), axis=1, keepdims=True)
        pooled = sums_ref[...] / jnp.maximum(cnt, 1.0)
        h = jnp.maximum(
            jnp.dot(pooled, W1_ref[...], preferred_element_type=jnp.float32)
            + b1_ref[...], 0.0)
        out_ref[...] = (
            jnp.dot(h, W2_ref[...], preferred_element_type=jnp.float32)
            + b2_ref[...])

    nblk = 8
    blk = BATCH // nblk
    return pl.pallas_call(
        body,
        grid=(nblk,),
        in_specs=[
            pl.BlockSpec((blk, SEQ_LEN), lambda i: (i, 0)),
            pl.BlockSpec((blk, EMBED_DIM), lambda i: (i, 0)),
            pl.BlockSpec(W1.shape, lambda i: (0, 0)),
            pl.BlockSpec(b1.shape, lambda i: (0, 0)),
            pl.BlockSpec(W2.shape, lambda i: (0, 0)),
            pl.BlockSpec(b2.shape, lambda i: (0, 0)),
        ],
        out_specs=pl.BlockSpec((blk, b2.shape[-1]), lambda i: (i, 0)),
        out_shape=jax.ShapeDtypeStruct((BATCH, b2.shape[-1]), jnp.float32),
    )(x, sums, W1, b1, W2, b2)


def kernel(x, emb, W1, b1, W2, b2):
    x = x.astype(jnp.int32)
    x3 = x.reshape(BATCH, 2, HALF)
    sums = _sc_pool_sums(x3, emb)
    return _tc_head(x, sums, W1, b1.reshape(1, -1), W2, b2.reshape(1, -1))
